# baseline (speedup 1.0000x reference)
import functools

import jax
import jax.numpy as jnp
from jax import lax
from jax.experimental import pallas as pl
from jax.experimental.pallas import tpu as pltpu
from jax.experimental.pallas import tpu_sc as plsc

_NC, _NS, _L = 2, 16, 16
_NW = _NC * _NS
_B = 16384
_ROWS_PER_C = _B // _NC
_ROWS_PER_W = _B // _NW


@functools.partial(
    pl.kernel,
    mesh=plsc.VectorSubcoreMesh(core_axis_name="c", subcore_axis_name="s"),
    compiler_params=pltpu.CompilerParams(needs_layout_passes=False),
    out_type=jax.ShapeDtypeStruct((_NW * _L,), jnp.float32),
    scratch_types=[
        pltpu.VMEM_SHARED((_ROWS_PER_C, 3), jnp.int32),
        pltpu.VMEM((_ROWS_PER_W, 3), jnp.int32),
        pltpu.VMEM((_L,), jnp.float32),
        pltpu.SemaphoreType.DMA,
    ],
)
def _sc_loss(x_hbm, tbl_hbm, out_hbm, xshared, xbuf, accbuf, sem):
    cid = lax.axis_index("c")
    sid = lax.axis_index("s")
    wid = cid * _NS + sid

    @pl.when(sid == 0)
    def _():
        pltpu.sync_copy(x_hbm.at[pl.ds(cid * _ROWS_PER_C, _ROWS_PER_C)], xshared)

    plsc.subcore_barrier()
    pltpu.sync_copy(xshared.at[pl.ds(sid * _ROWS_PER_W, _ROWS_PER_W)], xbuf)

    lanes = lax.iota(jnp.int32, _L)
    accbuf[...] = lanes.astype(jnp.float32) * 0.0
    pltpu.sync_copy(accbuf, out_hbm.at[pl.ds(wid * _L, _L)])


def kernel(x, table):
    partials = _sc_loss(x.astype(jnp.int32), table.astype(jnp.float32))
    return jnp.sum(partials)


# speedup vs baseline: 1.0816x; 1.0816x over previous
import functools

import jax
import jax.numpy as jnp
from jax import lax
from jax.experimental import pallas as pl
from jax.experimental.pallas import tpu as pltpu
from jax.experimental.pallas import tpu_sc as plsc

_NC, _NS, _L = 2, 16, 16
_NW = _NC * _NS
_B = 16384
_ROWS_PER_W = _B // _NW
_NCOMBO = 216
_NGRP = 16
_NCHUNK = 4
_CHUNK = _ROWS_PER_W // _NCHUNK

_TAKE_DNUMS = lax.GatherDimensionNumbers(
    offset_dims=(), collapsed_slice_dims=(0,), start_index_map=(0,))


def _take(vec, idx):
    return lax.gather(vec, idx[:, None], _TAKE_DNUMS, (1,),
                      mode=lax.GatherScatterMode.PROMISE_IN_BOUNDS)


@functools.partial(
    pl.kernel,
    mesh=plsc.VectorSubcoreMesh(core_axis_name="c", subcore_axis_name="s"),
    compiler_params=pltpu.CompilerParams(needs_layout_passes=False),
    out_type=jax.ShapeDtypeStruct((_NW * _L,), jnp.float32),
    scratch_types=[
        pltpu.VMEM((_ROWS_PER_W, 3), jnp.int32),
        pltpu.VMEM((_L,), jnp.int32),
        pltpu.VMEM((6, 10), jnp.float32),
        pltpu.VMEM((_NGRP * _L,), jnp.float32),
        pltpu.VMEM_SHARED((_NGRP * _L,), jnp.float32),
        pltpu.VMEM((_L,), jnp.float32),
        pltpu.SemaphoreType.DMA,
        pltpu.SemaphoreType.DMA,
        pltpu.SemaphoreType.DMA,
        pltpu.SemaphoreType.DMA,
        pltpu.SemaphoreType.DMA,
    ],
)
def _sc_loss(x_hbm, tbl_hbm, out_hbm, xbuf, czbuf, tbl, gbuf, gshared, accbuf,
             sem0, sem1, sem2, sem3, tsem):
    wid = lax.axis_index("s") * _NC + lax.axis_index("c")
    base = wid * _ROWS_PER_W
    xdmas = [
        pltpu.async_copy(
            x_hbm.at[pl.ds(base + k * _CHUNK, _CHUNK)],
            xbuf.at[pl.ds(k * _CHUNK, _CHUNK)], s)
        for k, s in enumerate((sem0, sem1, sem2, sem3))
    ]
    tdma = pltpu.async_copy(tbl_hbm, tbl, tsem)

    lanes = lax.iota(jnp.int32, _L)
    czbuf[...] = lanes * 0
    col0 = czbuf[...]

    vclamp = jnp.minimum(lanes, 5)
    tdma.wait()
    rows = [plsc.load_gather(tbl, [vclamp, col0 + d]) for d in range(10)]

    sid = lax.axis_index("s")
    c = jnp.minimum(lanes + sid * _L, _NCOMBO - 1)
    ch = c // 36
    rem = c - ch * 36
    cr = rem // 6
    ct = rem - cr * 6
    g = jnp.zeros((_L,), jnp.float32)
    for d in range(10):
        a = _take(rows[d], ch)
        b = _take(rows[d], cr)
        t = _take(rows[d], ct)
        g = g + jnp.maximum(1.0 - a - b + t, 0.0)
    accbuf[...] = g
    pltpu.sync_copy(accbuf, gshared.at[pl.ds(sid * _L, _L)])
    plsc.subcore_barrier()
    pltpu.sync_copy(gshared, gbuf)

    acc = jnp.zeros((_L,), jnp.float32)
    for i in range(_ROWS_PER_W // _L):
        if i % (_CHUNK // _L) == 0:
            xdmas[i // (_CHUNK // _L)].wait()
        ridx = lanes + i * _L
        h = plsc.load_gather(xbuf, [ridx, col0])
        r = plsc.load_gather(xbuf, [ridx, col0 + 1])
        t = plsc.load_gather(xbuf, [ridx, col0 + 2])
        code = h * 36 + r * 6 + t
        acc = acc + plsc.load_gather(gbuf, [code])
    accbuf[...] = acc
    pltpu.sync_copy(accbuf, out_hbm.at[pl.ds(wid * _L, _L)])


def kernel(x, table):
    partials = _sc_loss(x.astype(jnp.int32), table.astype(jnp.float32))
    return jnp.sum(partials)


# speedup vs baseline: 1.0940x; 1.0115x over previous
import functools

import jax
import jax.numpy as jnp
from jax import lax
from jax.experimental import pallas as pl
from jax.experimental.pallas import tpu as pltpu
from jax.experimental.pallas import tpu_sc as plsc

_NC, _NS, _L = 2, 16, 16
_NW = _NC * _NS
_B = 16384
_ROWS_PER_W = _B // _NW
_NCOMBO = 216
_NGRP = 16
_NCHUNK = 2
_CHUNK = _ROWS_PER_W // _NCHUNK

_TAKE_DNUMS = lax.GatherDimensionNumbers(
    offset_dims=(), collapsed_slice_dims=(0,), start_index_map=(0,))


def _take(vec, idx):
    return lax.gather(vec, idx[:, None], _TAKE_DNUMS, (1,),
                      mode=lax.GatherScatterMode.PROMISE_IN_BOUNDS)


@functools.partial(
    pl.kernel,
    mesh=plsc.VectorSubcoreMesh(core_axis_name="c", subcore_axis_name="s"),
    compiler_params=pltpu.CompilerParams(needs_layout_passes=False),
    out_type=jax.ShapeDtypeStruct((_NW * _L,), jnp.float32),
    scratch_types=[
        pltpu.VMEM((_ROWS_PER_W, 3), jnp.int32),
        pltpu.VMEM((_L,), jnp.int32),
        pltpu.VMEM((6, 10), jnp.float32),
        pltpu.VMEM((_NGRP * _L,), jnp.float32),
        pltpu.VMEM_SHARED((_NGRP * _L,), jnp.float32),
        pltpu.VMEM((_L,), jnp.float32),
        pltpu.SemaphoreType.DMA,
        pltpu.SemaphoreType.DMA,
        pltpu.SemaphoreType.DMA,
        pltpu.SemaphoreType.DMA,
        pltpu.SemaphoreType.DMA,
    ],
)
def _sc_loss(x_hbm, tbl_hbm, out_hbm, xbuf, czbuf, tbl, gbuf, gshared, accbuf,
             sem0, sem1, sem2, sem3, tsem):
    wid = lax.axis_index("s") * _NC + lax.axis_index("c")
    base = wid * _ROWS_PER_W
    xdmas = [
        pltpu.async_copy(
            x_hbm.at[pl.ds(base + k * _CHUNK, _CHUNK)],
            xbuf.at[pl.ds(k * _CHUNK, _CHUNK)], s)
        for k, s in enumerate((sem0, sem1, sem2, sem3)[:_NCHUNK])
    ]
    tdma = pltpu.async_copy(tbl_hbm, tbl, tsem)

    lanes = lax.iota(jnp.int32, _L)
    czbuf[...] = lanes * 0
    col0 = czbuf[...]

    vclamp = jnp.minimum(lanes, 5)
    tdma.wait()
    rows = [plsc.load_gather(tbl, [vclamp, col0 + d]) for d in range(10)]

    sid = lax.axis_index("s")
    c = jnp.minimum(lanes + sid * _L, _NCOMBO - 1)
    ch = c // 36
    rem = c - ch * 36
    cr = rem // 6
    ct = rem - cr * 6
    g = jnp.zeros((_L,), jnp.float32)
    for d in range(10):
        a = _take(rows[d], ch)
        b = _take(rows[d], cr)
        t = _take(rows[d], ct)
        g = g + jnp.maximum(1.0 - a - b + t, 0.0)
    accbuf[...] = g
    pltpu.sync_copy(accbuf, gshared.at[pl.ds(sid * _L, _L)])
    plsc.subcore_barrier()
    pltpu.sync_copy(gshared, gbuf)

    acc = jnp.zeros((_L,), jnp.float32)
    for i in range(_ROWS_PER_W // _L):
        if i % (_CHUNK // _L) == 0:
            xdmas[i // (_CHUNK // _L)].wait()
        ridx = lanes + i * _L
        h = plsc.load_gather(xbuf, [ridx, col0])
        r = plsc.load_gather(xbuf, [ridx, col0 + 1])
        t = plsc.load_gather(xbuf, [ridx, col0 + 2])
        code = h * 36 + r * 6 + t
        acc = acc + plsc.load_gather(gbuf, [code])
    accbuf[...] = acc
    pltpu.sync_copy(accbuf, out_hbm.at[pl.ds(wid * _L, _L)])


def kernel(x, table):
    partials = _sc_loss(x.astype(jnp.int32), table.astype(jnp.float32))
    return jnp.sum(partials)
